# baseline (speedup 1.0000x reference)
import functools
import jax
import jax.numpy as jnp
from jax import lax
from jax.experimental import pallas as pl
from jax.experimental.pallas import tpu as pltpu
from jax.experimental.pallas import tpu_sc as plsc

N = 4096
NW = 32
RPT = N // NW
G = 4
NGROUP = RPT // G
NWORD = N // 64
NR = N // 4


def _sc_body(x_hbm, m_hbm, o_hbm, xbuf, mbuf, obuf):
    c = lax.axis_index("c")
    s = lax.axis_index("s")
    wid = s * 2 + c
    iota = lax.iota(jnp.int32, 16)
    shifts = (iota & 3) * 8
    last = jnp.full((16,), 15, jnp.int32)
    dnums = lax.GatherDimensionNumbers(
        offset_dims=(), collapsed_slice_dims=(0,), start_index_map=(0,))

    def vgather(v, idx):
        return lax.gather(v, idx[:, None], dnums, slice_sizes=(1,),
                          mode=lax.GatherScatterMode.PROMISE_IN_BOUNDS)

    def do_group(g, _):
        row0 = wid * RPT + g * G
        pltpu.sync_copy(x_hbm.at[pl.ds(row0 * N, G * N)], xbuf)
        pltpu.sync_copy(m_hbm.at[pl.ds(row0 * NR, G * NR)], mbuf)

        def unit(i, carries):
            carries = list(carries)
            for r in range(G):
                w = mbuf[pl.ds(r * NR + i * 16, 16)]
                for cc in range(4):
                    base = r * N + i * 64 + cc * 16
                    xv = xbuf[pl.ds(base, 16)]
                    wsel = vgather(w, (iota >> 2) + 4 * cc)
                    mb = (wsel >> shifts) & 1
                    v = xv * mb.astype(jnp.float32)
                    cs = jnp.cumsum(v) + carries[r]
                    obuf[pl.ds(base, 16)] = cs
                    carries[r] = vgather(cs, last)
            return tuple(carries)

        zero = jnp.zeros((16,), jnp.float32)
        lax.fori_loop(0, NWORD, unit, (zero,) * G)
        pltpu.sync_copy(obuf, o_hbm.at[pl.ds(row0 * N, G * N)])
        return 0

    lax.fori_loop(0, NGROUP, do_group, 0)


def kernel(x, mask):
    mesh = plsc.VectorSubcoreMesh(core_axis_name="c", subcore_axis_name="s")
    f = functools.partial(
        pl.kernel, mesh=mesh,
        out_type=jax.ShapeDtypeStruct((N * N,), jnp.float32),
        scratch_types=[
            pltpu.VMEM((G * N,), jnp.float32),
            pltpu.VMEM((G * NR,), jnp.int32),
            pltpu.VMEM((G * N,), jnp.float32),
        ],
        compiler_params=pltpu.CompilerParams(needs_layout_passes=False),
    )(_sc_body)
    mw = lax.bitcast_convert_type(
        mask.view(jnp.uint8).reshape(N * NR, 4), jnp.int32)
    return f(x.reshape(N * N), mw).reshape(N, N)


# speedup vs baseline: 6.1079x; 6.1079x over previous
import functools
import jax
import jax.numpy as jnp
from jax import lax
from jax.experimental import pallas as pl
from jax.experimental.pallas import tpu as pltpu
from jax.experimental.pallas import tpu_sc as plsc

N = 4096
NW = 32
RPT = N // NW
G = 8
NGROUP = RPT // G
NWORD = N // 64
NR = N // 4


def _sc_body(x_hbm, m_hbm, o_hbm, xbuf, mbuf, obuf):
    c = lax.axis_index("c")
    s = lax.axis_index("s")
    wid = s * 2 + c
    iota = lax.iota(jnp.int32, 16)
    shifts = (iota & 3) * 8
    last = jnp.full((16,), 15, jnp.int32)
    dnums = lax.GatherDimensionNumbers(
        offset_dims=(), collapsed_slice_dims=(0,), start_index_map=(0,))

    def vgather(v, idx):
        return lax.gather(v, idx[:, None], dnums, slice_sizes=(1,),
                          mode=lax.GatherScatterMode.PROMISE_IN_BOUNDS)

    def do_group(g, _):
        row0 = wid * RPT + g * G
        pltpu.sync_copy(x_hbm.at[pl.ds(row0, G)], xbuf)
        pltpu.sync_copy(m_hbm.at[pl.ds(row0, G)], mbuf)

        def unit(i, carries):
            carries = list(carries)
            for r in range(G):
                w = mbuf[r, pl.ds(i * 16, 16)]
                for cc in range(4):
                    col = i * 64 + cc * 16
                    xv = xbuf[r, pl.ds(col, 16)]
                    wsel = vgather(w, (iota >> 2) + 4 * cc)
                    mb = (wsel >> shifts) & 1
                    v = xv * mb.astype(jnp.float32)
                    cs = jnp.cumsum(v) + carries[r]
                    obuf[r, pl.ds(col, 16)] = cs
                    carries[r] = vgather(cs, last)
            return tuple(carries)

        zero = jnp.zeros((16,), jnp.float32)
        lax.fori_loop(0, NWORD, unit, (zero,) * G)
        pltpu.sync_copy(obuf, o_hbm.at[pl.ds(row0, G)])
        return 0

    lax.fori_loop(0, NGROUP, do_group, 0)


def kernel(x, mask):
    mesh = plsc.VectorSubcoreMesh(core_axis_name="c", subcore_axis_name="s")
    f = functools.partial(
        pl.kernel, mesh=mesh,
        out_type=jax.ShapeDtypeStruct((N, N), jnp.float32),
        scratch_types=[
            pltpu.VMEM((G, N), jnp.float32),
            pltpu.VMEM((G, NR), jnp.int32),
            pltpu.VMEM((G, N), jnp.float32),
        ],
        compiler_params=pltpu.CompilerParams(
            needs_layout_passes=False, use_tc_tiling_on_sc=True),
    )(_sc_body)
    mw = lax.bitcast_convert_type(
        mask.view(jnp.uint8).reshape(N, NR, 4), jnp.int32)
    return f(x, mw)


# speedup vs baseline: 42.1829x; 6.9063x over previous
import jax
import jax.numpy as jnp
from jax.experimental import pallas as pl
from jax.experimental.pallas import tpu as pltpu

N = 4096
R = 2048
C = 512


def _scan_kernel(x_ref, m_ref, o_ref, carry_ref):
    j = pl.program_id(1)

    @pl.when(j == 0)
    def _():
        carry_ref[...] = jnp.zeros_like(carry_ref)

    xm = jnp.where(m_ref[...], x_ref[...], 0.0)
    row = jax.lax.broadcasted_iota(jnp.int32, (C, C), 0)
    col = jax.lax.broadcasted_iota(jnp.int32, (C, C), 1)
    tri = (row <= col).astype(jnp.float32)
    cs = jax.lax.dot(xm, tri, precision=jax.lax.Precision.DEFAULT,
                     preferred_element_type=jnp.float32)
    out = cs + carry_ref[...]
    o_ref[...] = out
    carry_ref[...] = out[:, C - 1:C]


def kernel(x, mask):
    grid = (N // R, N // C)
    return pl.pallas_call(
        _scan_kernel,
        grid=grid,
        in_specs=[
            pl.BlockSpec((R, C), lambda i, j: (i, j)),
            pl.BlockSpec((R, C), lambda i, j: (i, j)),
        ],
        out_specs=pl.BlockSpec((R, C), lambda i, j: (i, j)),
        out_shape=jax.ShapeDtypeStruct((N, N), jnp.float32),
        scratch_shapes=[pltpu.VMEM((R, 1), jnp.float32)],
        compiler_params=pltpu.CompilerParams(
            dimension_semantics=("parallel", "arbitrary")),
    )(x, mask)


# speedup vs baseline: 42.3362x; 1.0036x over previous
import jax
import jax.numpy as jnp
from jax.experimental import pallas as pl
from jax.experimental.pallas import tpu as pltpu

N = 4096
R = 4096
C = 512


def _scan_kernel(x_ref, m_ref, o_ref, carry_ref):
    j = pl.program_id(1)

    @pl.when(j == 0)
    def _():
        carry_ref[...] = jnp.zeros_like(carry_ref)

    xm = jnp.where(m_ref[...], x_ref[...], 0.0)
    row = jax.lax.broadcasted_iota(jnp.int32, (C, C), 0)
    col = jax.lax.broadcasted_iota(jnp.int32, (C, C), 1)
    tri = (row <= col).astype(jnp.float32)
    cs = jax.lax.dot(xm, tri, precision=jax.lax.Precision.DEFAULT,
                     preferred_element_type=jnp.float32)
    out = cs + carry_ref[...]
    o_ref[...] = out
    carry_ref[...] = out[:, C - 1:C]


def kernel(x, mask):
    grid = (N // R, N // C)
    return pl.pallas_call(
        _scan_kernel,
        grid=grid,
        in_specs=[
            pl.BlockSpec((R, C), lambda i, j: (i, j)),
            pl.BlockSpec((R, C), lambda i, j: (i, j)),
        ],
        out_specs=pl.BlockSpec((R, C), lambda i, j: (i, j)),
        out_shape=jax.ShapeDtypeStruct((N, N), jnp.float32),
        scratch_shapes=[pltpu.VMEM((R, 1), jnp.float32)],
        compiler_params=pltpu.CompilerParams(
            dimension_semantics=("parallel", "arbitrary")),
    )(x, mask)
